# baseline (speedup 1.0000x reference)
import functools

import jax
import jax.numpy as jnp
from jax import lax
from jax.experimental import pallas as pl
from jax.experimental.pallas import tpu as pltpu
from jax.experimental.pallas import tpu_sc as plsc

_KE = 64
_CHB = 24
_RC = 32


def _build(N, D, E, NC, NT, L):
  chalf = D // NC
  rpt = -(-(N + 1) // (NT * _RC)) * _RC
  n_pad = NT * rpt
  nrow_ch = rpt // _RC
  vpr = chalf // L
  CH = -(-(E + N) // (NT * _KE))
  CH = -(-CH // _CHB) * _CHB
  nblk = CH // _CHB
  etot = NT * CH * _KE

  mesh = plsc.VectorSubcoreMesh(core_axis_name="c", subcore_axis_name="s",
                                num_cores=NC, num_subcores=NT)

  @functools.partial(
      pl.kernel,
      out_type=(
          jax.ShapeDtypeStruct((n_pad, D), jnp.float32),
          jax.ShapeDtypeStruct((n_pad, D), jnp.float32),
      ),
      mesh=mesh,
      compiler_params=pltpu.CompilerParams(needs_layout_passes=False),
      scratch_types=[
          pltpu.VMEM((_CHB, _KE), jnp.int32),
          pltpu.VMEM((_CHB, _KE), jnp.int32),
          pltpu.VMEM((_KE, chalf), jnp.float32),
          pltpu.VMEM((_KE, chalf), jnp.float32),
          pltpu.VMEM((_KE, chalf), jnp.float32),
          pltpu.VMEM((_RC, chalf), jnp.float32),
          pltpu.VMEM((_RC, chalf), jnp.float32),
          pltpu.VMEM((_RC // 2, chalf), jnp.float32),
          pltpu.VMEM((rpt,), jnp.float32),
          pltpu.VMEM((rpt,), jnp.float32),
          pltpu.VMEM((rpt,), jnp.float32),
          pltpu.VMEM_SHARED((n_pad, chalf), jnp.float32),
          pltpu.SemaphoreType.DMA,
          pltpu.SemaphoreType.DMA,
          pltpu.SemaphoreType.DMA,
          pltpu.SemaphoreType.DMA,
          pltpu.SemaphoreType.DMA,
          pltpu.SemaphoreType.DMA,
      ],
  )
  def lightgcn(cols_ref, rows_ref, x0_ref, a_ref, y_ref,
               cidx, ridx, g0, g1, g2, sbuf, abuf, zbuf, degp, disb,
               dis2b, s_acc, gsem0, gsem1, gsem2, ssem0, ssem1, ssem2):
    c = lax.axis_index("c")
    s = lax.axis_index("s")
    base = pl.multiple_of(s * rpt, _RC)
    coff = pl.multiple_of(c * chalf, chalf)
    colsl = pl.ds(coff, chalf)

    @pl.loop(0, _RC // 2)
    def _(r):
      for k in range(vpr):
        zbuf[r, pl.ds(L * k, L)] = jnp.zeros((L,), jnp.float32)

    def zero_sacc(r0):
      h = _RC // 2
      pltpu.sync_copy(zbuf, s_acc.at[pl.ds(r0, h)])
      pltpu.sync_copy(zbuf, s_acc.at[pl.ds(r0 + h, h)])

    @pl.loop(0, rpt // L)
    def _(i):
      degp[pl.ds(pl.multiple_of(L * i, L), L)] = jnp.zeros((L,), jnp.float32)

    ones = jnp.ones((L,), jnp.float32)

    @pl.loop(0, NT * nblk)
    def _(tb):
      t = tb // nblk
      b = tb % nblk
      bsl = pl.ds(pl.multiple_of(b * _CHB, _CHB), _CHB)
      pltpu.sync_copy(cols_ref.at[t, bsl], cidx)

      @pl.loop(0, _CHB)
      def _(j):
        for k in range(_KE // L):
          idx = cidx[j, pl.ds(L * k, L)]
          loc = idx - base
          m = (loc >= 0) & (loc < rpt)
          locs = jnp.where(m, loc, 0)
          plsc.addupdate_scatter(degp, [locs], ones, mask=m)

    @pl.loop(0, rpt // L)
    def _(i):
      sl = pl.ds(pl.multiple_of(L * i, L), L)
      d = degp[sl]
      bits = plsc.bitcast(d, jnp.int32)
      y = plsc.bitcast(jnp.int32(0x5F3759DF) - (bits >> 1), jnp.float32)
      for _ in range(3):
        y = y * (1.5 - 0.5 * d * y * y)
      dis = jnp.where(d > 0.0, y, 0.0)
      disb[sl] = dis
      dis2b[sl] = dis * dis

    @pl.loop(0, nrow_ch)
    def _(i):
      rowsl = pl.ds(pl.multiple_of(base + _RC * i, _RC), _RC)
      pltpu.sync_copy(x0_ref.at[rowsl, colsl], sbuf)
      pltpu.sync_copy(sbuf, a_ref.at[rowsl, colsl])

      @pl.loop(0, _RC)
      def _(r):
        dv = plsc.load_gather(disb, [jnp.full((L,), _RC * i + r, jnp.int32)])
        for k in range(vpr):
          sl = pl.ds(L * k, L)
          sbuf[r, sl] = sbuf[r, sl] * dv

      pltpu.sync_copy(sbuf, y_ref.at[rowsl, colsl])
      zero_sacc(pl.multiple_of(base + _RC * i, _RC))

    plsc.subcore_barrier()

    def gsrc(j):
      return y_ref.at[cidx.at[j], colsl]

    for layer in range(3):
      last = layer == 2

      bufs = (g0, g1, g2)
      gsems = (gsem0, gsem1, gsem2)
      ssems = (ssem0, ssem1, ssem2)
      nb = len(bufs)

      def sdst(j):
        return s_acc.at[ridx.at[j]]

      @pl.loop(0, nblk)
      def _(b):
        bsl = pl.ds(pl.multiple_of(b * _CHB, _CHB), _CHB)
        pltpu.sync_copy(cols_ref.at[s, bsl], cidx)
        pltpu.sync_copy(rows_ref.at[s, bsl], ridx)
        for l in range(nb):
          pltpu.async_copy(gsrc(l), bufs[l], gsems[l])

        @pl.loop(0, _CHB // nb - 1)
        def _(it):
          j0 = nb * it
          for l in range(nb):
            pltpu.make_async_copy(gsrc(j0 + l), bufs[l], gsems[l]).wait()
            pltpu.async_copy(bufs[l], sdst(j0 + l), ssems[l], add=True)
          for l in range(nb):
            pltpu.make_async_copy(bufs[l], sdst(j0 + l), ssems[l]).wait()
            pltpu.async_copy(gsrc(j0 + nb + l), bufs[l], gsems[l])

        j0 = _CHB - nb
        for l in range(nb):
          pltpu.make_async_copy(gsrc(j0 + l), bufs[l], gsems[l]).wait()
          pltpu.async_copy(bufs[l], sdst(j0 + l), ssems[l], add=True)
        for l in range(nb):
          pltpu.make_async_copy(bufs[l], sdst(j0 + l), ssems[l]).wait()

      plsc.subcore_barrier()

      @pl.loop(0, nrow_ch)
      def _(i):
        r0 = pl.multiple_of(base + _RC * i, _RC)
        rowsl = pl.ds(r0, _RC)
        pltpu.sync_copy(s_acc.at[rowsl], sbuf)
        if not last:
          zero_sacc(r0)
        pltpu.sync_copy(a_ref.at[rowsl, colsl], abuf)

        @pl.loop(0, _RC)
        def _(r):
          ri = jnp.full((L,), _RC * i + r, jnp.int32)
          dv = plsc.load_gather(disb, [ri])
          d2 = plsc.load_gather(dis2b, [ri])
          for k in range(vpr):
            sl = pl.ds(L * k, L)
            sv = sbuf[r, sl]
            av = abuf[r, sl] + dv * sv
            if last:
              av = av * 0.25
            abuf[r, sl] = av
            if not last:
              sbuf[r, sl] = d2 * sv

        pltpu.sync_copy(abuf, a_ref.at[rowsl, colsl])
        if not last:
          pltpu.sync_copy(sbuf, y_ref.at[rowsl, colsl])

      plsc.subcore_barrier()

  return lightgcn, CH, rpt, n_pad, etot


def kernel(edge_index, embedding_weight):
  N, D = embedding_weight.shape
  E = edge_index.shape[1]
  info = plsc.get_sparse_core_info()
  NC, NT, L = info.num_cores, info.num_subcores, info.num_lanes
  fn, CH, rpt, n_pad, etot = _build(N, D, E, NC, NT, L)

  loop = jnp.arange(N, dtype=jnp.int32)
  npad_e = etot - E - N
  trash = jnp.full((npad_e,), N, jnp.int32)
  row = jnp.concatenate([edge_index[0].astype(jnp.int32), loop, trash])
  col = jnp.concatenate([edge_index[1].astype(jnp.int32), loop, trash])
  cols3 = col.reshape(NT, CH, _KE)
  rows3 = row.reshape(NT, CH, _KE)
  x0p = jnp.zeros((n_pad, D), jnp.float32).at[:N].set(embedding_weight)
  a, _ = fn(cols3, rows3, x0p)
  return a[:N]


# speedup vs baseline: 1.1392x; 1.1392x over previous
import functools

import jax
import jax.numpy as jnp
from jax import lax
from jax.experimental import pallas as pl
from jax.experimental.pallas import tpu as pltpu
from jax.experimental.pallas import tpu_sc as plsc

_KE = 32
_NB = 4
_CHB = 24
_RC = 32


def _build(N, D, E, NC, NT, L):
  chalf = D // NC
  rpt = -(-(N + 1) // (NT * _RC)) * _RC
  n_pad = NT * rpt
  nrow_ch = rpt // _RC
  vpr = chalf // L
  CH = -(-(E + N) // (NT * _KE))
  CH = -(-CH // _CHB) * _CHB
  nblk = CH // _CHB
  etot = NT * CH * _KE
  dgr = n_pad // chalf
  dgs = rpt // chalf
  csh = chalf.bit_length() - 1
  assert chalf == 1 << csh

  mesh = plsc.VectorSubcoreMesh(core_axis_name="c", subcore_axis_name="s",
                                num_cores=NC, num_subcores=NT)

  @functools.partial(
      pl.kernel,
      out_type=(
          jax.ShapeDtypeStruct((n_pad, D), jnp.float32),
          jax.ShapeDtypeStruct((n_pad, D), jnp.float32),
      ),
      mesh=mesh,
      compiler_params=pltpu.CompilerParams(needs_layout_passes=False),
      scratch_types=[
          pltpu.VMEM((_CHB, _KE), jnp.int32),
          pltpu.VMEM((_CHB, _KE), jnp.int32),
          *[pltpu.VMEM((_KE, chalf), jnp.float32) for _ in range(_NB)],
          pltpu.VMEM((_RC, chalf), jnp.float32),
          pltpu.VMEM((_RC, chalf), jnp.float32),
          pltpu.VMEM((_RC // 2, chalf), jnp.float32),
          pltpu.VMEM((dgr, chalf), jnp.float32),
          pltpu.VMEM((NT * dgs,), jnp.int32),
          pltpu.VMEM((rpt,), jnp.float32),
          pltpu.VMEM((rpt,), jnp.float32),
          pltpu.VMEM_SHARED((n_pad, chalf), jnp.float32),
          *[pltpu.SemaphoreType.DMA for _ in range(2 * _NB)],
      ],
  )
  def lightgcn(cols_ref, rows_ref, x0_ref, a_ref, y_ref, cidx, ridx, *rest):
    bufs = rest[:_NB]
    sbuf, abuf, zbuf, degp, didx, disb, dis2b, s_acc = rest[_NB:_NB + 8]
    gsems = rest[_NB + 8:_NB + 8 + _NB]
    ssems = rest[_NB + 8 + _NB:]
    c = lax.axis_index("c")
    s = lax.axis_index("s")
    base = pl.multiple_of(s * rpt, _RC)
    coff = pl.multiple_of(c * chalf, chalf)
    colsl = pl.ds(coff, chalf)
    iota = lax.iota(jnp.int32, L)

    @pl.loop(0, _RC // 2)
    def _(r):
      for k in range(vpr):
        zbuf[r, pl.ds(L * k, L)] = jnp.zeros((L,), jnp.float32)

    @pl.loop(0, dgr)
    def _(r):
      for k in range(vpr):
        degp[r, pl.ds(L * k, L)] = jnp.zeros((L,), jnp.float32)

    def zero_sacc(r0):
      h = _RC // 2
      pltpu.sync_copy(zbuf, s_acc.at[pl.ds(r0, h)])
      pltpu.sync_copy(zbuf, s_acc.at[pl.ds(r0 + h, h)])

    ones = jnp.ones((L,), jnp.float32)

    @pl.loop(0, nblk)
    def _(b):
      bsl = pl.ds(pl.multiple_of(b * _CHB, _CHB), _CHB)
      pltpu.sync_copy(cols_ref.at[s, bsl], cidx)

      @pl.loop(0, _CHB)
      def _(j):
        for k in range(_KE // L):
          idx = cidx[j, pl.ds(L * k, L)]
          plsc.addupdate_scatter(degp, [idx >> csh, idx & (chalf - 1)], ones)

    pltpu.sync_copy(degp, s_acc.at[pl.ds(pl.multiple_of(s * dgr, 8), dgr)])

    @pl.loop(0, (NT * dgs) // L)
    def _(v):
      i = L * v + iota
      t = i // dgs
      r = i - t * dgs
      didx[pl.ds(pl.multiple_of(L * v, L), L)] = t * dgr + dgs * s + r

    plsc.subcore_barrier()
    pltpu.sync_copy(s_acc.at[didx], degp)

    @pl.loop(1, NT)
    def _(t):
      for r in range(dgs):
        for k in range(vpr):
          sl = pl.ds(L * k, L)
          degp[r, sl] = degp[r, sl] + degp[dgs * t + r, sl]

    @pl.loop(0, rpt // L)
    def _(i):
      flat = L * i + iota
      d = plsc.load_gather(degp, [flat >> csh, flat & (chalf - 1)])
      bits = plsc.bitcast(d, jnp.int32)
      y = plsc.bitcast(jnp.int32(0x5F3759DF) - (bits >> 1), jnp.float32)
      for _ in range(3):
        y = y * (1.5 - 0.5 * d * y * y)
      dis = jnp.where(d > 0.0, y, 0.0)
      sl = pl.ds(pl.multiple_of(L * i, L), L)
      disb[sl] = dis
      dis2b[sl] = dis * dis

    plsc.subcore_barrier()

    @pl.loop(0, nrow_ch)
    def _(i):
      r0 = pl.multiple_of(base + _RC * i, _RC)
      rowsl = pl.ds(r0, _RC)
      pltpu.sync_copy(x0_ref.at[rowsl, colsl], sbuf)
      pltpu.sync_copy(sbuf, a_ref.at[rowsl, colsl])

      @pl.loop(0, _RC)
      def _(r):
        dv = plsc.load_gather(disb, [jnp.full((L,), _RC * i + r, jnp.int32)])
        for k in range(vpr):
          sl = pl.ds(L * k, L)
          sbuf[r, sl] = sbuf[r, sl] * dv

      pltpu.sync_copy(sbuf, y_ref.at[rowsl, colsl])
      zero_sacc(r0)

    plsc.subcore_barrier()

    def gsrc(j):
      return y_ref.at[cidx.at[j], colsl]

    def sdst(j):
      return s_acc.at[ridx.at[j]]

    for layer in range(3):
      last = layer == 2

      @pl.loop(0, nblk)
      def _(b):
        bsl = pl.ds(pl.multiple_of(b * _CHB, _CHB), _CHB)
        pltpu.sync_copy(cols_ref.at[s, bsl], cidx)
        pltpu.sync_copy(rows_ref.at[s, bsl], ridx)
        for l in range(_NB):
          pltpu.async_copy(gsrc(l), bufs[l], gsems[l])

        @pl.loop(0, _CHB // _NB - 1)
        def _(it):
          j0 = _NB * it
          for l in range(_NB):
            pltpu.make_async_copy(gsrc(j0 + l), bufs[l], gsems[l]).wait()
            pltpu.async_copy(bufs[l], sdst(j0 + l), ssems[l], add=True)
          for l in range(_NB):
            pltpu.make_async_copy(bufs[l], sdst(j0 + l), ssems[l]).wait()
            pltpu.async_copy(gsrc(j0 + _NB + l), bufs[l], gsems[l])

        j0 = _CHB - _NB
        for l in range(_NB):
          pltpu.make_async_copy(gsrc(j0 + l), bufs[l], gsems[l]).wait()
          pltpu.async_copy(bufs[l], sdst(j0 + l), ssems[l], add=True)
        for l in range(_NB):
          pltpu.make_async_copy(bufs[l], sdst(j0 + l), ssems[l]).wait()

      plsc.subcore_barrier()

      @pl.loop(0, nrow_ch)
      def _(i):
        r0 = pl.multiple_of(base + _RC * i, _RC)
        rowsl = pl.ds(r0, _RC)
        pltpu.sync_copy(s_acc.at[rowsl], sbuf)
        if not last:
          zero_sacc(r0)
        pltpu.sync_copy(a_ref.at[rowsl, colsl], abuf)

        @pl.loop(0, _RC)
        def _(r):
          ri = jnp.full((L,), _RC * i + r, jnp.int32)
          dv = plsc.load_gather(disb, [ri])
          d2 = plsc.load_gather(dis2b, [ri])
          for k in range(vpr):
            sl = pl.ds(L * k, L)
            sv = sbuf[r, sl]
            av = abuf[r, sl] + dv * sv
            if last:
              av = av * 0.25
            abuf[r, sl] = av
            if not last:
              sbuf[r, sl] = d2 * sv

        pltpu.sync_copy(abuf, a_ref.at[rowsl, colsl])
        if not last:
          pltpu.sync_copy(sbuf, y_ref.at[rowsl, colsl])

      plsc.subcore_barrier()

  return lightgcn, CH, rpt, n_pad, etot


def kernel(edge_index, embedding_weight):
  N, D = embedding_weight.shape
  E = edge_index.shape[1]
  info = plsc.get_sparse_core_info()
  NC, NT, L = info.num_cores, info.num_subcores, info.num_lanes
  fn, CH, rpt, n_pad, etot = _build(N, D, E, NC, NT, L)

  loop = jnp.arange(N, dtype=jnp.int32)
  npad_e = etot - E - N
  trash = jnp.full((npad_e,), N, jnp.int32)
  row = jnp.concatenate([edge_index[0].astype(jnp.int32), loop, trash])
  col = jnp.concatenate([edge_index[1].astype(jnp.int32), loop, trash])
  cols3 = col.reshape(NT, CH, _KE)
  rows3 = row.reshape(NT, CH, _KE)
  x0p = jnp.zeros((n_pad, D), jnp.float32).at[:N].set(embedding_weight)
  a, _ = fn(cols3, rows3, x0p)
  return a[:N]


# speedup vs baseline: 1.2253x; 1.0755x over previous
import functools

import jax
import jax.numpy as jnp
from jax import lax
from jax.experimental import pallas as pl
from jax.experimental.pallas import tpu as pltpu
from jax.experimental.pallas import tpu_sc as plsc

_KE = 32
_NB = 4
_CHB = 24
_RC = 32


def _build(N, D, E, NC, NT, L):
  chalf = D // NC
  rpt = -(-(N + 1) // (NT * _RC)) * _RC
  n_pad = NT * rpt
  nrow_ch = rpt // _RC
  vpr = chalf // L
  CH = -(-(E + N) // (NT * _KE))
  CH = -(-CH // _CHB) * _CHB
  nblk = CH // _CHB
  etot = NT * CH * _KE
  dgr = n_pad // chalf
  dgs = rpt // chalf
  csh = chalf.bit_length() - 1
  assert chalf == 1 << csh

  mesh = plsc.VectorSubcoreMesh(core_axis_name="c", subcore_axis_name="s",
                                num_cores=NC, num_subcores=NT)

  @functools.partial(
      pl.kernel,
      out_type=(
          jax.ShapeDtypeStruct((n_pad, D), jnp.float32),
          jax.ShapeDtypeStruct((n_pad, D), jnp.float32),
      ),
      mesh=mesh,
      compiler_params=pltpu.CompilerParams(needs_layout_passes=False),
      scratch_types=[
          pltpu.VMEM((_CHB, _KE), jnp.int32),
          pltpu.VMEM((_CHB, _KE), jnp.int32),
          *[pltpu.VMEM((_KE, chalf), jnp.float32) for _ in range(_NB)],
          pltpu.VMEM((_RC // 2, chalf), jnp.float32),
          pltpu.VMEM((dgr, chalf), jnp.float32),
          pltpu.VMEM((NT * dgs,), jnp.int32),
          pltpu.VMEM((rpt,), jnp.float32),
          pltpu.VMEM((rpt,), jnp.float32),
          pltpu.VMEM_SHARED((n_pad, chalf), jnp.float32),
          *[pltpu.SemaphoreType.DMA for _ in range(2 * _NB)],
      ],
  )
  def lightgcn(cols_ref, rows_ref, x0_ref, a_ref, y_ref, cidx, ridx, *rest):
    bufs = rest[:_NB]
    zbuf, degp, didx, disb, dis2b, s_acc = rest[_NB:_NB + 6]
    gsems = rest[_NB + 6:_NB + 6 + _NB]
    ssems = rest[_NB + 6 + _NB:]
    c = lax.axis_index("c")
    s = lax.axis_index("s")
    base = pl.multiple_of(s * rpt, _RC)
    coff = pl.multiple_of(c * chalf, chalf)
    colsl = pl.ds(coff, chalf)
    iota = lax.iota(jnp.int32, L)

    @pl.loop(0, _RC // 2)
    def _(r):
      for k in range(vpr):
        zbuf[r, pl.ds(L * k, L)] = jnp.zeros((L,), jnp.float32)

    @pl.loop(0, dgr)
    def _(r):
      for k in range(vpr):
        degp[r, pl.ds(L * k, L)] = jnp.zeros((L,), jnp.float32)

    ones = jnp.ones((L,), jnp.float32)

    @pl.loop(0, nblk)
    def _(b):
      bsl = pl.ds(pl.multiple_of(b * _CHB, _CHB), _CHB)
      pltpu.sync_copy(cols_ref.at[s, bsl], cidx)

      @pl.loop(0, _CHB)
      def _(j):
        for k in range(_KE // L):
          idx = cidx[j, pl.ds(L * k, L)]
          plsc.addupdate_scatter(degp, [idx >> csh, idx & (chalf - 1)], ones)

    pltpu.sync_copy(degp, s_acc.at[pl.ds(pl.multiple_of(s * dgr, 8), dgr)])

    @pl.loop(0, (NT * dgs) // L)
    def _(v):
      i = L * v + iota
      t = i // dgs
      r = i - t * dgs
      didx[pl.ds(pl.multiple_of(L * v, L), L)] = t * dgr + dgs * s + r

    plsc.subcore_barrier()
    pltpu.sync_copy(s_acc.at[didx], degp)

    @pl.loop(1, NT)
    def _(t):
      for r in range(dgs):
        for k in range(vpr):
          sl = pl.ds(L * k, L)
          degp[r, sl] = degp[r, sl] + degp[dgs * t + r, sl]

    @pl.loop(0, rpt // L)
    def _(i):
      flat = L * i + iota
      d = plsc.load_gather(degp, [flat >> csh, flat & (chalf - 1)])
      bits = plsc.bitcast(d, jnp.int32)
      y = plsc.bitcast(jnp.int32(0x5F3759DF) - (bits >> 1), jnp.float32)
      for _ in range(3):
        y = y * (1.5 - 0.5 * d * y * y)
      dis = jnp.where(d > 0.0, y, 0.0)
      sl = pl.ds(pl.multiple_of(L * i, L), L)
      disb[sl] = dis
      dis2b[sl] = dis * dis

    plsc.subcore_barrier()

    def rslice(i):
      return pl.ds(pl.multiple_of(base + _RC * i, _RC), _RC)

    def zslices(i):
      h = _RC // 2
      r0 = pl.multiple_of(base + _RC * i, h)
      return (pl.ds(r0, h), pl.ds(r0 + h, h))

    def zero_async(i):
      for zs in zslices(i):
        pltpu.async_copy(zbuf, s_acc.at[zs], ssems[0])

    def zero_drain():
      @pl.loop(0, 2 * nrow_ch)
      def _(_i):
        pltpu.make_async_copy(zbuf, s_acc.at[pl.ds(base, _RC // 2)],
                              ssems[0]).wait()

    pltpu.async_copy(x0_ref.at[rslice(0), colsl], bufs[0], gsems[0])

    @pl.loop(0, nrow_ch // 2)
    def _(r2):
      for par in range(2):
        i = 2 * r2 + par
        xin, gs = bufs[par], gsems[par]

        @pl.when(i + 1 < nrow_ch)
        def _():
          pltpu.async_copy(x0_ref.at[rslice(i + 1), colsl],
                           bufs[1 - par], gsems[1 - par])

        pltpu.make_async_copy(x0_ref.at[rslice(i), colsl], xin, gs).wait()
        pltpu.sync_copy(xin, a_ref.at[rslice(i), colsl])
        zero_async(i)

        @pl.loop(0, _RC)
        def _(r):
          dv = plsc.load_gather(disb,
                                [jnp.full((L,), _RC * i + r, jnp.int32)])
          for k in range(vpr):
            sl = pl.ds(L * k, L)
            xin[r, sl] = xin[r, sl] * dv

        pltpu.sync_copy(xin, y_ref.at[rslice(i), colsl])

    zero_drain()
    plsc.subcore_barrier()

    def gsrc(j):
      return y_ref.at[cidx.at[j], colsl]

    def sdst(j):
      return s_acc.at[ridx.at[j]]

    for layer in range(3):
      last = layer == 2

      @pl.loop(0, nblk)
      def _(b):
        bsl = pl.ds(pl.multiple_of(b * _CHB, _CHB), _CHB)
        pltpu.sync_copy(cols_ref.at[s, bsl], cidx)
        pltpu.sync_copy(rows_ref.at[s, bsl], ridx)
        for l in range(_NB):
          pltpu.async_copy(gsrc(l), bufs[l], gsems[l])

        @pl.loop(0, _CHB // _NB - 1)
        def _(it):
          j0 = _NB * it
          for l in range(_NB):
            pltpu.make_async_copy(gsrc(j0 + l), bufs[l], gsems[l]).wait()
            pltpu.async_copy(bufs[l], sdst(j0 + l), ssems[l], add=True)
          for l in range(_NB):
            pltpu.make_async_copy(bufs[l], sdst(j0 + l), ssems[l]).wait()
            pltpu.async_copy(gsrc(j0 + _NB + l), bufs[l], gsems[l])

        j0 = _CHB - _NB
        for l in range(_NB):
          pltpu.make_async_copy(gsrc(j0 + l), bufs[l], gsems[l]).wait()
          pltpu.async_copy(bufs[l], sdst(j0 + l), ssems[l], add=True)
        for l in range(_NB):
          pltpu.make_async_copy(bufs[l], sdst(j0 + l), ssems[l]).wait()

      plsc.subcore_barrier()

      pltpu.async_copy(s_acc.at[rslice(0)], bufs[0], gsems[0])
      pltpu.async_copy(a_ref.at[rslice(0), colsl], bufs[2], gsems[2])

      @pl.loop(0, nrow_ch // 2)
      def _(r2):
        for par in range(2):
          i = 2 * r2 + par
          sin, ain = bufs[par], bufs[2 + par]
          gss, gsa = gsems[par], gsems[2 + par]

          @pl.when(i + 1 < nrow_ch)
          def _():
            pltpu.async_copy(s_acc.at[rslice(i + 1)],
                             bufs[1 - par], gsems[1 - par])
            pltpu.async_copy(a_ref.at[rslice(i + 1), colsl],
                             bufs[3 - par], gsems[3 - par])

          pltpu.make_async_copy(s_acc.at[rslice(i)], sin, gss).wait()
          pltpu.make_async_copy(a_ref.at[rslice(i), colsl], ain, gsa).wait()
          if not last:
            zero_async(i)

          @pl.loop(0, _RC)
          def _(r):
            ri = jnp.full((L,), _RC * i + r, jnp.int32)
            dv = plsc.load_gather(disb, [ri])
            d2 = plsc.load_gather(dis2b, [ri])
            for k in range(vpr):
              sl = pl.ds(L * k, L)
              sv = sin[r, sl]
              av = ain[r, sl] + dv * sv
              if last:
                av = av * 0.25
              ain[r, sl] = av
              if not last:
                sin[r, sl] = d2 * sv

          pltpu.sync_copy(ain, a_ref.at[rslice(i), colsl])
          if not last:
            pltpu.sync_copy(sin, y_ref.at[rslice(i), colsl])

      if not last:
        zero_drain()
      plsc.subcore_barrier()

  return lightgcn, CH, rpt, n_pad, etot


def kernel(edge_index, embedding_weight):
  N, D = embedding_weight.shape
  E = edge_index.shape[1]
  info = plsc.get_sparse_core_info()
  NC, NT, L = info.num_cores, info.num_subcores, info.num_lanes
  fn, CH, rpt, n_pad, etot = _build(N, D, E, NC, NT, L)

  loop = jnp.arange(N, dtype=jnp.int32)
  npad_e = etot - E - N
  trash = jnp.full((npad_e,), N, jnp.int32)
  row = jnp.concatenate([edge_index[0].astype(jnp.int32), loop, trash])
  col = jnp.concatenate([edge_index[1].astype(jnp.int32), loop, trash])
  cols3 = col.reshape(NT, CH, _KE)
  rows3 = row.reshape(NT, CH, _KE)
  x0p = jnp.zeros((n_pad, D), jnp.float32).at[:N].set(embedding_weight)
  a, _ = fn(cols3, rows3, x0p)
  return a[:N]


# speedup vs baseline: 1.2723x; 1.0384x over previous
import functools

import jax
import jax.numpy as jnp
from jax import lax
from jax.experimental import pallas as pl
from jax.experimental.pallas import tpu as pltpu
from jax.experimental.pallas import tpu_sc as plsc

_KE = 32
_NB = 4
_CHB = 48
_RC = 32


def _build(N, D, E, NC, NT, L):
  chalf = D // NC
  rpt = -(-(N + 1) // (NT * _RC)) * _RC
  n_pad = NT * rpt
  nrow_ch = rpt // _RC
  vpr = chalf // L
  CH = -(-(E + N) // (NT * _KE))
  CH = -(-CH // _CHB) * _CHB
  nblk = CH // _CHB
  etot = NT * CH * _KE
  dgr = n_pad // chalf
  dgs = rpt // chalf
  csh = chalf.bit_length() - 1
  assert chalf == 1 << csh

  mesh = plsc.VectorSubcoreMesh(core_axis_name="c", subcore_axis_name="s",
                                num_cores=NC, num_subcores=NT)

  @functools.partial(
      pl.kernel,
      out_type=(
          jax.ShapeDtypeStruct((n_pad, D), jnp.float32),
          jax.ShapeDtypeStruct((n_pad, D), jnp.float32),
      ),
      mesh=mesh,
      compiler_params=pltpu.CompilerParams(needs_layout_passes=False),
      scratch_types=[
          pltpu.VMEM((_CHB, _KE), jnp.int32),
          pltpu.VMEM((_CHB, _KE), jnp.int32),
          *[pltpu.VMEM((_KE, chalf), jnp.float32) for _ in range(_NB)],
          pltpu.VMEM((_RC // 2, chalf), jnp.float32),
          pltpu.VMEM((dgr, chalf), jnp.float32),
          pltpu.VMEM((NT * dgs,), jnp.int32),
          pltpu.VMEM((rpt,), jnp.float32),
          pltpu.VMEM((rpt,), jnp.float32),
          pltpu.VMEM_SHARED((n_pad, chalf), jnp.float32),
          *[pltpu.SemaphoreType.DMA for _ in range(2 * _NB)],
      ],
  )
  def lightgcn(cols_ref, rows_ref, x0_ref, a_ref, y_ref, cidx, ridx, *rest):
    bufs = rest[:_NB]
    zbuf, degp, didx, disb, dis2b, s_acc = rest[_NB:_NB + 6]
    gsems = rest[_NB + 6:_NB + 6 + _NB]
    ssems = rest[_NB + 6 + _NB:]
    c = lax.axis_index("c")
    s = lax.axis_index("s")
    base = pl.multiple_of(s * rpt, _RC)
    coff = pl.multiple_of(c * chalf, chalf)
    colsl = pl.ds(coff, chalf)
    iota = lax.iota(jnp.int32, L)

    @pl.loop(0, _RC // 2)
    def _(r):
      for k in range(vpr):
        zbuf[r, pl.ds(L * k, L)] = jnp.zeros((L,), jnp.float32)

    @pl.loop(0, dgr)
    def _(r):
      for k in range(vpr):
        degp[r, pl.ds(L * k, L)] = jnp.zeros((L,), jnp.float32)

    ones = jnp.ones((L,), jnp.float32)

    @pl.loop(0, nblk)
    def _(b):
      bsl = pl.ds(pl.multiple_of(b * _CHB, _CHB), _CHB)
      pltpu.sync_copy(cols_ref.at[s, bsl], cidx)

      @pl.loop(0, _CHB)
      def _(j):
        for k in range(_KE // L):
          idx = cidx[j, pl.ds(L * k, L)]
          plsc.addupdate_scatter(degp, [idx >> csh, idx & (chalf - 1)], ones)

    pltpu.sync_copy(degp, s_acc.at[pl.ds(pl.multiple_of(s * dgr, 8), dgr)])

    @pl.loop(0, (NT * dgs) // L)
    def _(v):
      i = L * v + iota
      t = i // dgs
      r = i - t * dgs
      didx[pl.ds(pl.multiple_of(L * v, L), L)] = t * dgr + dgs * s + r

    plsc.subcore_barrier()
    pltpu.sync_copy(s_acc.at[didx], degp)

    @pl.loop(1, NT)
    def _(t):
      for r in range(dgs):
        for k in range(vpr):
          sl = pl.ds(L * k, L)
          degp[r, sl] = degp[r, sl] + degp[dgs * t + r, sl]

    @pl.loop(0, rpt // L)
    def _(i):
      flat = L * i + iota
      d = plsc.load_gather(degp, [flat >> csh, flat & (chalf - 1)])
      bits = plsc.bitcast(d, jnp.int32)
      y = plsc.bitcast(jnp.int32(0x5F3759DF) - (bits >> 1), jnp.float32)
      for _ in range(3):
        y = y * (1.5 - 0.5 * d * y * y)
      dis = jnp.where(d > 0.0, y, 0.0)
      sl = pl.ds(pl.multiple_of(L * i, L), L)
      disb[sl] = dis
      dis2b[sl] = dis * dis

    plsc.subcore_barrier()

    def rslice(i):
      return pl.ds(pl.multiple_of(base + _RC * i, _RC), _RC)

    def zslices(i):
      h = _RC // 2
      r0 = pl.multiple_of(base + _RC * i, h)
      return (pl.ds(r0, h), pl.ds(r0 + h, h))

    def zero_async(i):
      for zs in zslices(i):
        pltpu.async_copy(zbuf, s_acc.at[zs], ssems[0])

    def zero_drain():
      @pl.loop(0, 2 * nrow_ch)
      def _(_i):
        pltpu.make_async_copy(zbuf, s_acc.at[pl.ds(base, _RC // 2)],
                              ssems[0]).wait()

    pltpu.async_copy(x0_ref.at[rslice(0), colsl], bufs[0], gsems[0])

    @pl.loop(0, nrow_ch // 2)
    def _(r2):
      for par in range(2):
        i = 2 * r2 + par
        xin, gs = bufs[par], gsems[par]

        @pl.when(i + 1 < nrow_ch)
        def _():
          pltpu.async_copy(x0_ref.at[rslice(i + 1), colsl],
                           bufs[1 - par], gsems[1 - par])

        pltpu.make_async_copy(x0_ref.at[rslice(i), colsl], xin, gs).wait()
        pltpu.sync_copy(xin, a_ref.at[rslice(i), colsl])
        zero_async(i)

        @pl.loop(0, _RC)
        def _(r):
          dv = plsc.load_gather(disb,
                                [jnp.full((L,), _RC * i + r, jnp.int32)])
          for k in range(vpr):
            sl = pl.ds(L * k, L)
            xin[r, sl] = xin[r, sl] * dv

        pltpu.sync_copy(xin, y_ref.at[rslice(i), colsl])

    zero_drain()
    plsc.subcore_barrier()

    def gsrc(j):
      return y_ref.at[cidx.at[j], colsl]

    def sdst(j):
      return s_acc.at[ridx.at[j]]

    for layer in range(3):
      last = layer == 2

      @pl.loop(0, nblk)
      def _(b):
        bsl = pl.ds(pl.multiple_of(b * _CHB, _CHB), _CHB)
        pltpu.sync_copy(cols_ref.at[s, bsl], cidx)
        pltpu.sync_copy(rows_ref.at[s, bsl], ridx)
        for l in range(_NB):
          pltpu.async_copy(gsrc(l), bufs[l], gsems[l])

        @pl.loop(0, _CHB // _NB - 1)
        def _(it):
          j0 = _NB * it
          for l in range(_NB):
            pltpu.make_async_copy(gsrc(j0 + l), bufs[l], gsems[l]).wait()
            pltpu.async_copy(bufs[l], sdst(j0 + l), ssems[l], add=True)
          for l in range(_NB):
            pltpu.make_async_copy(bufs[l], sdst(j0 + l), ssems[l]).wait()
            pltpu.async_copy(gsrc(j0 + _NB + l), bufs[l], gsems[l])

        j0 = _CHB - _NB
        for l in range(_NB):
          pltpu.make_async_copy(gsrc(j0 + l), bufs[l], gsems[l]).wait()
          pltpu.async_copy(bufs[l], sdst(j0 + l), ssems[l], add=True)
        for l in range(_NB):
          pltpu.make_async_copy(bufs[l], sdst(j0 + l), ssems[l]).wait()

      plsc.subcore_barrier()

      pltpu.async_copy(s_acc.at[rslice(0)], bufs[0], gsems[0])
      pltpu.async_copy(a_ref.at[rslice(0), colsl], bufs[2], gsems[2])

      @pl.loop(0, nrow_ch // 2)
      def _(r2):
        for par in range(2):
          i = 2 * r2 + par
          sin, ain = bufs[par], bufs[2 + par]
          gss, gsa = gsems[par], gsems[2 + par]

          @pl.when(i + 1 < nrow_ch)
          def _():
            pltpu.async_copy(s_acc.at[rslice(i + 1)],
                             bufs[1 - par], gsems[1 - par])
            pltpu.async_copy(a_ref.at[rslice(i + 1), colsl],
                             bufs[3 - par], gsems[3 - par])

          pltpu.make_async_copy(s_acc.at[rslice(i)], sin, gss).wait()
          pltpu.make_async_copy(a_ref.at[rslice(i), colsl], ain, gsa).wait()
          if not last:
            zero_async(i)

          @pl.loop(0, _RC)
          def _(r):
            ri = jnp.full((L,), _RC * i + r, jnp.int32)
            dv = plsc.load_gather(disb, [ri])
            d2 = plsc.load_gather(dis2b, [ri])
            for k in range(vpr):
              sl = pl.ds(L * k, L)
              sv = sin[r, sl]
              av = ain[r, sl] + dv * sv
              if last:
                av = av * 0.25
              ain[r, sl] = av
              if not last:
                sin[r, sl] = d2 * sv

          pltpu.sync_copy(ain, a_ref.at[rslice(i), colsl])
          if not last:
            pltpu.sync_copy(sin, y_ref.at[rslice(i), colsl])

      if not last:
        zero_drain()
      plsc.subcore_barrier()

  return lightgcn, CH, rpt, n_pad, etot


def kernel(edge_index, embedding_weight):
  N, D = embedding_weight.shape
  E = edge_index.shape[1]
  info = plsc.get_sparse_core_info()
  NC, NT, L = info.num_cores, info.num_subcores, info.num_lanes
  fn, CH, rpt, n_pad, etot = _build(N, D, E, NC, NT, L)

  loop = jnp.arange(N, dtype=jnp.int32)
  npad_e = etot - E - N
  trash = jnp.full((npad_e,), N, jnp.int32)
  row = jnp.concatenate([edge_index[0].astype(jnp.int32), loop, trash])
  col = jnp.concatenate([edge_index[1].astype(jnp.int32), loop, trash])
  cols3 = col.reshape(NT, CH, _KE)
  rows3 = row.reshape(NT, CH, _KE)
  x0p = jnp.zeros((n_pad, D), jnp.float32).at[:N].set(embedding_weight)
  a, _ = fn(cols3, rows3, x0p)
  return a[:N]


# speedup vs baseline: 1.2743x; 1.0016x over previous
import functools

import jax
import jax.numpy as jnp
from jax import lax
from jax.experimental import pallas as pl
from jax.experimental.pallas import tpu as pltpu
from jax.experimental.pallas import tpu_sc as plsc

_KE = 32
_NB = 4
_CHB = 48
_RC = 32


def _build(N, D, E, NC, NT, L):
  chalf = D // NC
  rpt = -(-(N + 1) // (NT * _RC)) * _RC
  n_pad = NT * rpt
  nrow_ch = rpt // _RC
  vpr = chalf // L
  CH = -(-(E + N) // (NT * _KE))
  CH = -(-CH // _CHB) * _CHB
  nblk = CH // _CHB
  etot = NT * CH * _KE
  dgr = n_pad // chalf
  dgs = rpt // chalf
  csh = chalf.bit_length() - 1
  assert chalf == 1 << csh

  mesh = plsc.VectorSubcoreMesh(core_axis_name="c", subcore_axis_name="s",
                                num_cores=NC, num_subcores=NT)

  @functools.partial(
      pl.kernel,
      out_type=(
          jax.ShapeDtypeStruct((n_pad, D), jnp.float32),
          jax.ShapeDtypeStruct((n_pad, D), jnp.float32),
      ),
      mesh=mesh,
      compiler_params=pltpu.CompilerParams(needs_layout_passes=False),
      scratch_types=[
          pltpu.VMEM((_CHB, _KE), jnp.int32),
          pltpu.VMEM((_CHB, _KE), jnp.int32),
          *[pltpu.VMEM((_KE, chalf), jnp.float32) for _ in range(_NB)],
          pltpu.VMEM((_RC // 2, chalf), jnp.float32),
          pltpu.VMEM((dgr, chalf), jnp.float32),
          pltpu.VMEM((NT * dgs,), jnp.int32),
          pltpu.VMEM((rpt,), jnp.float32),
          pltpu.VMEM((rpt,), jnp.float32),
          pltpu.VMEM_SHARED((n_pad, chalf), jnp.float32),
          *[pltpu.SemaphoreType.DMA for _ in range(2 * _NB)],
      ],
  )
  def lightgcn(cols_ref, rows_ref, x0_ref, a_ref, y_ref, cidx, ridx, *rest):
    bufs = rest[:_NB]
    zbuf, degp, didx, disb, dis2b, s_acc = rest[_NB:_NB + 6]
    gsems = rest[_NB + 6:_NB + 6 + _NB]
    ssems = rest[_NB + 6 + _NB:]
    c = lax.axis_index("c")
    s = lax.axis_index("s")
    base = pl.multiple_of(s * rpt, _RC)
    coff = pl.multiple_of(c * chalf, chalf)
    colsl = pl.ds(coff, chalf)
    iota = lax.iota(jnp.int32, L)

    @pl.loop(0, _RC // 2)
    def _(r):
      for k in range(vpr):
        zbuf[r, pl.ds(L * k, L)] = jnp.zeros((L,), jnp.float32)

    @pl.loop(0, dgr)
    def _(r):
      for k in range(vpr):
        degp[r, pl.ds(L * k, L)] = jnp.zeros((L,), jnp.float32)

    ones = jnp.ones((L,), jnp.float32)

    @pl.loop(0, nblk)
    def _(b):
      bsl = pl.ds(pl.multiple_of(b * _CHB, _CHB), _CHB)
      pltpu.sync_copy(cols_ref.at[s, bsl], cidx)

      @pl.loop(0, _CHB)
      def _(j):
        for k in range(_KE // L):
          idx = cidx[j, pl.ds(L * k, L)]
          plsc.addupdate_scatter(degp, [idx >> csh, idx & (chalf - 1)], ones)

    pltpu.sync_copy(degp, s_acc.at[pl.ds(pl.multiple_of(s * dgr, 8), dgr)])

    @pl.loop(0, (NT * dgs) // L)
    def _(v):
      i = L * v + iota
      t = i // dgs
      r = i - t * dgs
      didx[pl.ds(pl.multiple_of(L * v, L), L)] = t * dgr + dgs * s + r

    plsc.subcore_barrier()
    pltpu.sync_copy(s_acc.at[didx], degp)

    @pl.loop(1, NT)
    def _(t):
      for r in range(dgs):
        for k in range(vpr):
          sl = pl.ds(L * k, L)
          degp[r, sl] = degp[r, sl] + degp[dgs * t + r, sl]

    @pl.loop(0, rpt // L)
    def _(i):
      flat = L * i + iota
      d = plsc.load_gather(degp, [flat >> csh, flat & (chalf - 1)])
      bits = plsc.bitcast(d, jnp.int32)
      y = plsc.bitcast(jnp.int32(0x5F3759DF) - (bits >> 1), jnp.float32)
      for _ in range(3):
        y = y * (1.5 - 0.5 * d * y * y)
      dis = jnp.where(d > 0.0, y, 0.0)
      sl = pl.ds(pl.multiple_of(L * i, L), L)
      disb[sl] = dis
      dis2b[sl] = dis * dis

    plsc.subcore_barrier()

    def rslice(i):
      return pl.ds(pl.multiple_of(base + _RC * i, _RC), _RC)

    def zslices(i):
      h = _RC // 2
      r0 = pl.multiple_of(base + _RC * i, h)
      return (pl.ds(r0, h), pl.ds(r0 + h, h))

    def zero_async(i):
      for zs in zslices(i):
        pltpu.async_copy(zbuf, s_acc.at[zs], ssems[0])

    def zero_drain():
      @pl.loop(0, 2 * nrow_ch)
      def _(_i):
        pltpu.make_async_copy(zbuf, s_acc.at[pl.ds(base, _RC // 2)],
                              ssems[0]).wait()

    pltpu.async_copy(x0_ref.at[rslice(0), colsl], bufs[0], gsems[0])

    @pl.loop(0, nrow_ch // 2)
    def _(r2):
      for par in range(2):
        i = 2 * r2 + par
        xin, gs = bufs[par], gsems[par]

        @pl.when(i + 1 < nrow_ch)
        def _():
          pltpu.async_copy(x0_ref.at[rslice(i + 1), colsl],
                           bufs[1 - par], gsems[1 - par])

        pltpu.make_async_copy(x0_ref.at[rslice(i), colsl], xin, gs).wait()
        pltpu.sync_copy(xin, a_ref.at[rslice(i), colsl])
        zero_async(i)

        @pl.loop(0, _RC)
        def _(r):
          dv = plsc.load_gather(disb,
                                [jnp.full((L,), _RC * i + r, jnp.int32)])
          for k in range(vpr):
            sl = pl.ds(L * k, L)
            xin[r, sl] = xin[r, sl] * dv

        pltpu.sync_copy(xin, y_ref.at[rslice(i), colsl])

    zero_drain()
    plsc.subcore_barrier()

    def gsrc(j):
      return y_ref.at[cidx.at[j], colsl]

    def sdst(j):
      return s_acc.at[ridx.at[j]]

    for layer in range(3):
      last = layer == 2

      @pl.loop(0, nblk)
      def _(b):
        bsl = pl.ds(pl.multiple_of(b * _CHB, _CHB), _CHB)
        pltpu.sync_copy(cols_ref.at[s, bsl], cidx)
        pltpu.sync_copy(rows_ref.at[s, bsl], ridx)
        for l in range(_NB):
          pltpu.async_copy(gsrc(l), bufs[l], gsems[l])

        @pl.loop(0, _CHB // _NB - 1)
        def _(it):
          j0 = _NB * it
          for l in range(_NB):
            pltpu.make_async_copy(gsrc(j0 + l), bufs[l], gsems[l]).wait()
            pltpu.async_copy(bufs[l], sdst(j0 + l), ssems[l], add=True)
          for l in range(_NB):
            pltpu.make_async_copy(bufs[l], sdst(j0 + l), ssems[l]).wait()
            pltpu.async_copy(gsrc(j0 + _NB + l), bufs[l], gsems[l])

        j0 = _CHB - _NB
        for l in range(_NB):
          pltpu.make_async_copy(gsrc(j0 + l), bufs[l], gsems[l]).wait()
          pltpu.async_copy(bufs[l], sdst(j0 + l), ssems[l], add=True)
        for l in range(_NB):
          pltpu.make_async_copy(bufs[l], sdst(j0 + l), ssems[l]).wait()

      plsc.subcore_barrier()

      pltpu.async_copy(s_acc.at[rslice(0)], bufs[0], gsems[0])
      pltpu.async_copy(a_ref.at[rslice(0), colsl], bufs[2], gsems[2])

      @pl.loop(0, nrow_ch // 2)
      def _(r2):
        for par in range(2):
          i = 2 * r2 + par
          sin, ain = bufs[par], bufs[2 + par]
          gss, gsa = gsems[par], gsems[2 + par]

          @pl.when(i >= 1)
          def _():
            pltpu.make_async_copy(bufs[3 - par],
                                  a_ref.at[rslice(i - 1), colsl],
                                  ssems[1]).wait()
            if not last:
              pltpu.make_async_copy(bufs[1 - par],
                                    y_ref.at[rslice(i - 1), colsl],
                                    ssems[2]).wait()

          @pl.when(i + 1 < nrow_ch)
          def _():
            pltpu.async_copy(s_acc.at[rslice(i + 1)],
                             bufs[1 - par], gsems[1 - par])
            pltpu.async_copy(a_ref.at[rslice(i + 1), colsl],
                             bufs[3 - par], gsems[3 - par])

          pltpu.make_async_copy(s_acc.at[rslice(i)], sin, gss).wait()
          pltpu.make_async_copy(a_ref.at[rslice(i), colsl], ain, gsa).wait()
          if not last:
            zero_async(i)

          @pl.loop(0, _RC)
          def _(r):
            ri = jnp.full((L,), _RC * i + r, jnp.int32)
            dv = plsc.load_gather(disb, [ri])
            d2 = plsc.load_gather(dis2b, [ri])
            for k in range(vpr):
              sl = pl.ds(L * k, L)
              sv = sin[r, sl]
              av = ain[r, sl] + dv * sv
              if last:
                av = av * 0.25
              ain[r, sl] = av
              if not last:
                sin[r, sl] = d2 * sv

          pltpu.async_copy(ain, a_ref.at[rslice(i), colsl], ssems[1])
          if not last:
            pltpu.async_copy(sin, y_ref.at[rslice(i), colsl], ssems[2])

      li = nrow_ch - 1
      pltpu.make_async_copy(bufs[2 + (li % 2)], a_ref.at[rslice(li), colsl],
                            ssems[1]).wait()
      if not last:
        pltpu.make_async_copy(bufs[li % 2], y_ref.at[rslice(li), colsl],
                              ssems[2]).wait()
        zero_drain()
      plsc.subcore_barrier()

  return lightgcn, CH, rpt, n_pad, etot


def kernel(edge_index, embedding_weight):
  N, D = embedding_weight.shape
  E = edge_index.shape[1]
  info = plsc.get_sparse_core_info()
  NC, NT, L = info.num_cores, info.num_subcores, info.num_lanes
  fn, CH, rpt, n_pad, etot = _build(N, D, E, NC, NT, L)

  loop = jnp.arange(N, dtype=jnp.int32)
  npad_e = etot - E - N
  trash = jnp.full((npad_e,), N, jnp.int32)
  row = jnp.concatenate([edge_index[0].astype(jnp.int32), loop, trash])
  col = jnp.concatenate([edge_index[1].astype(jnp.int32), loop, trash])
  cols3 = col.reshape(NT, CH, _KE)
  rows3 = row.reshape(NT, CH, _KE)
  x0p = jnp.zeros((n_pad, D), jnp.float32).at[:N].set(embedding_weight)
  a, _ = fn(cols3, rows3, x0p)
  return a[:N]
